# Initial kernel scaffold; baseline (speedup 1.0000x reference)
import jax
import jax.numpy as jnp
from jax.experimental import pallas as pl


def kernel(embedding_A, embedding_B):
    raise NotImplementedError("write your pallas kernel here")


# speedup vs baseline: 12.5663x; 12.5663x over previous
import jax
import jax.numpy as jnp
from jax.experimental import pallas as pl

_B = 4096
_D = 32
_BLK = 256
_GRID = _B // _BLK
_K = 129
_TOP = 0.95
_BOT = 0.05
_INV_T = 1.0 / 0.07
_FILL_W = 512
_BISECT_ITERS = 36


def _body(a_ref, b_ref, out_ref):
    pid = pl.program_id(0)
    a = a_ref[...]
    b = b_ref[...]
    an = a / jnp.maximum(jnp.sqrt(jnp.sum(a * a, axis=1, keepdims=True)), 1e-12)
    bn = b / jnp.maximum(jnp.sqrt(jnp.sum(b * b, axis=1, keepdims=True)), 1e-12)
    sim = jax.lax.dot_general(
        an, bn, (((1,), (1,)), ((), ())),
        preferred_element_type=jnp.float32,
        precision=jax.lax.Precision.HIGHEST)

    col = jax.lax.broadcasted_iota(jnp.int32, sim.shape, 1)
    row = jax.lax.broadcasted_iota(jnp.int32, sim.shape, 0) + pid * _BLK
    diag = col == row
    pos = jnp.sum(jnp.where(diag, sim, 0.0), axis=1, keepdims=True)

    m = (sim >= _BOT * pos) & (sim <= _TOP * pos) & jnp.logical_not(diag)
    n = jnp.sum(m.astype(jnp.float32), axis=1, keepdims=True)
    e = jnp.exp(sim * _INV_T)
    epos = jnp.exp(pos * _INV_T)

    def _step(_, carry):
        lo, hi = carry
        mid = 0.5 * (lo + hi)
        c = jnp.sum(jnp.where(m & (sim >= mid), 1.0, 0.0), axis=1, keepdims=True)
        ge = c >= _K
        return jnp.where(ge, mid, lo), jnp.where(ge, hi, mid)

    lo0 = _BOT * pos
    hi0 = pos
    t, _ = jax.lax.fori_loop(0, _BISECT_ITERS, _step, (lo0, hi0))
    gt = m & (sim > t)
    c_gt = jnp.sum(jnp.where(gt, 1.0, 0.0), axis=1, keepdims=True)
    s_a = (jnp.sum(jnp.where(gt, e, 0.0), axis=1, keepdims=True)
           + (_K - c_gt) * jnp.exp(t * _INV_T) + epos)

    nm = jnp.where(m[:, :_FILL_W], 0.0, 1.0)
    kk = jax.lax.broadcasted_iota(jnp.int32, (_FILL_W, _FILL_W), 0)
    jj = jax.lax.broadcasted_iota(jnp.int32, (_FILL_W, _FILL_W), 1)
    tri = jnp.where(kk <= jj, 1.0, 0.0)
    cum = jax.lax.dot_general(
        nm, tri, (((1,), (0,)), ((), ())),
        preferred_element_type=jnp.float32,
        precision=jax.lax.Precision.HIGHEST)
    fill = (nm > 0.0) & (cum <= (_K - n))
    s_b = (jnp.sum(jnp.where(m | diag, e, 0.0), axis=1, keepdims=True)
           + jnp.sum(jnp.where(fill & jnp.logical_not(diag[:, :_FILL_W]),
                               e[:, :_FILL_W], 0.0), axis=1, keepdims=True))

    s = jnp.where(n >= _K, s_a, s_b)
    loss = jnp.log(s) - pos * _INV_T
    block_total = jnp.sum(loss)

    @pl.when(pid == 0)
    def _():
        out_ref[...] = jnp.zeros_like(out_ref)

    out_ref[...] += block_total.reshape(1, 1)


def kernel(embedding_A, embedding_B):
    total = pl.pallas_call(
        _body,
        grid=(_GRID,),
        in_specs=[
            pl.BlockSpec((_BLK, _D), lambda i: (i, 0)),
            pl.BlockSpec((_B, _D), lambda i: (0, 0)),
        ],
        out_specs=pl.BlockSpec((1, 1), lambda i: (0, 0)),
        out_shape=jax.ShapeDtypeStruct((1, 1), jnp.float32),
    )(embedding_A, embedding_B)
    return total[0, 0] / _B


# speedup vs baseline: 32.0344x; 2.5492x over previous
import jax
import jax.numpy as jnp
from jax.experimental import pallas as pl
from jax.experimental.pallas import tpu as pltpu

_B = 4096
_D = 32
_BLK = 256
_GRID = _B // _BLK
_K = 129
_TOP = 0.95
_BOT = 0.05
_INV_T = 1.0 / 0.07
_FILL_W = 512
_BISECT_ITERS = 18


def _body(a_ref, b_ref, out_ref, bn_ref):
    pid = pl.program_id(0)

    @pl.when(pid == 0)
    def _():
        b = b_ref[...]
        bn_ref[...] = b / jnp.maximum(
            jnp.sqrt(jnp.sum(b * b, axis=1, keepdims=True)), 1e-12)

    a = a_ref[...]
    an = a / jnp.maximum(jnp.sqrt(jnp.sum(a * a, axis=1, keepdims=True)), 1e-12)
    bn = bn_ref[...]
    sim = jax.lax.dot_general(
        an, bn, (((1,), (1,)), ((), ())),
        preferred_element_type=jnp.float32)

    bnb = bn_ref[pl.ds(pid * _BLK, _BLK), :]
    pos = jnp.sum(an * bnb, axis=1, keepdims=True)

    col = jax.lax.broadcasted_iota(jnp.int32, sim.shape, 1)
    row = jax.lax.broadcasted_iota(jnp.int32, sim.shape, 0) + pid * _BLK
    diag = col == row
    m = (sim >= _BOT * pos) & (sim <= _TOP * pos) & jnp.logical_not(diag)
    vm = jnp.where(m, sim, -2.0)
    n = jnp.sum(jnp.where(m, 1.0, 0.0), axis=1, keepdims=True)
    e = jnp.exp(sim * _INV_T)
    epos = jnp.exp(pos * _INV_T)

    lo = _BOT * pos
    hi = pos
    for _ in range(_BISECT_ITERS):
        mid = 0.5 * (lo + hi)
        c = jnp.sum(jnp.where(vm >= mid, 1.0, 0.0), axis=1, keepdims=True)
        ge = c >= _K
        lo = jnp.where(ge, mid, lo)
        hi = jnp.where(ge, hi, mid)
    t = lo
    gt = vm > t
    c_gt = jnp.sum(jnp.where(gt, 1.0, 0.0), axis=1, keepdims=True)
    s_a = (jnp.sum(jnp.where(gt, e, 0.0), axis=1, keepdims=True)
           + (_K - c_gt) * jnp.exp(t * _INV_T) + epos)

    nm = jnp.where(m[:, :_FILL_W], 0.0, 1.0)
    kk = jax.lax.broadcasted_iota(jnp.int32, (_FILL_W, _FILL_W), 0)
    ll = jax.lax.broadcasted_iota(jnp.int32, (_FILL_W, _FILL_W), 1)
    tri = jnp.where(kk <= ll, 1.0, 0.0)
    cum = jax.lax.dot_general(
        nm, tri, (((1,), (0,)), ((), ())),
        preferred_element_type=jnp.float32,
        precision=jax.lax.Precision.HIGHEST)
    fill = (nm > 0.0) & (cum <= (_K - n))
    s_b = (jnp.sum(jnp.where(m | diag, e, 0.0), axis=1, keepdims=True)
           + jnp.sum(jnp.where(fill & jnp.logical_not(diag[:, :_FILL_W]),
                               e[:, :_FILL_W], 0.0), axis=1, keepdims=True))

    s = jnp.where(n >= _K, s_a, s_b)
    loss = jnp.log(s) - pos * _INV_T
    block_total = jnp.sum(loss)

    @pl.when(pid == 0)
    def _():
        out_ref[...] = jnp.zeros_like(out_ref)

    out_ref[...] += block_total.reshape(1, 1)


def kernel(embedding_A, embedding_B):
    total = pl.pallas_call(
        _body,
        grid=(_GRID,),
        in_specs=[
            pl.BlockSpec((_BLK, _D), lambda i: (i, 0)),
            pl.BlockSpec((_B, _D), lambda i: (0, 0)),
        ],
        out_specs=pl.BlockSpec((1, 1), lambda i: (0, 0)),
        out_shape=jax.ShapeDtypeStruct((1, 1), jnp.float32),
        scratch_shapes=[pltpu.VMEM((_B, _D), jnp.float32)],
    )(embedding_A, embedding_B)
    return total[0, 0] / _B
